# baseline (speedup 1.0000x reference)
import jax
import jax.numpy as jnp
from jax import lax
from jax.experimental import pallas as pl
from jax.experimental.pallas import tpu as pltpu

N = 10000
D = 256
B = 1024

_RB = 256
_HB = _RB // 2
_NBLK = B // _RB


def _body(idx_ref, a_hbm, emb_hbm, emb_ref, w1_ref, w2t_ref, b_ref,
          out_ref, a_buf, cen_buf, t_scr, emb_bf_scr, sems):
    i = pl.program_id(0)
    slot = lax.rem(i, 2)

    def copies(block, slot_, half):
        descs = []
        for r in range(half * _HB, (half + 1) * _HB):
            g = idx_ref[block * _RB + r]
            descs.append(pltpu.make_async_copy(
                a_hbm.at[pl.ds(g, 1)],
                a_buf.at[slot_, pl.ds(r, 1)],
                sems.at[slot_, half]))
            descs.append(pltpu.make_async_copy(
                emb_hbm.at[pl.ds(g, 1)],
                cen_buf.at[slot_, pl.ds(r, 1)],
                sems.at[slot_, half]))
        return descs

    def issue(block, slot_):
        for half in (0, 1):
            for d in copies(block, slot_, half):
                d.start()

    def drain(block, slot_, half):
        for d in copies(block, slot_, half):
            d.wait()

    @pl.when(i == 0)
    def _prologue():
        issue(0, 0)
        emb_bf_scr[...] = emb_ref[...].astype(jnp.bfloat16)
        t_scr[...] = lax.dot_general(
            w2t_ref[...], emb_bf_scr[...], (((1,), (1,)), ((), ())),
            preferred_element_type=jnp.float32) + b_ref[0, 0]

    @pl.when(i + 1 < _NBLK)
    def _prefetch_next():
        issue(i + 1, 1 - slot)

    for half in (0, 1):
        drain(i, slot, half)
        rows = pl.ds(half * _HB, _HB)
        cen = cen_buf[slot, rows]
        c_col = jnp.dot(cen, w1_ref[...],
                        preferred_element_type=jnp.float32)
        s = c_col + t_scr[...]
        s = jnp.maximum(s, 0.2 * s)
        w = (a_buf[slot, rows] * s).astype(jnp.bfloat16)
        out_ref[rows] = jnp.dot(w, emb_bf_scr[...],
                                preferred_element_type=jnp.float32) + cen


def _aggregate(node_indexes, a, emb, w1, w2t, b2d):
    grid_spec = pltpu.PrefetchScalarGridSpec(
        num_scalar_prefetch=1,
        grid=(_NBLK,),
        in_specs=[
            pl.BlockSpec(memory_space=pl.ANY),
            pl.BlockSpec(memory_space=pl.ANY),
            pl.BlockSpec((N, D), lambda i, idx: (0, 0)),
            pl.BlockSpec((D, 1), lambda i, idx: (0, 0)),
            pl.BlockSpec((1, D), lambda i, idx: (0, 0)),
            pl.BlockSpec((1, 1), lambda i, idx: (0, 0)),
        ],
        out_specs=pl.BlockSpec((_RB, D), lambda i, idx: (i, 0)),
        scratch_shapes=[
            pltpu.VMEM((2, _RB, N), jnp.float32),
            pltpu.VMEM((2, _RB, D), jnp.float32),
            pltpu.VMEM((1, N), jnp.float32),
            pltpu.VMEM((N, D), jnp.bfloat16),
            pltpu.SemaphoreType.DMA((2, 2)),
        ],
    )
    return pl.pallas_call(
        _body,
        grid_spec=grid_spec,
        out_shape=jax.ShapeDtypeStruct((B, D), jnp.float32),
        compiler_params=pltpu.CompilerParams(
            vmem_limit_bytes=100 * 1024 * 1024),
    )(node_indexes, a, emb, emb, w1, w2t, b2d)


def kernel(node_indexes, A, embedding_states, W_fc, b_fc):
    w1 = W_fc[:D]
    w2t = W_fc[D:].reshape(1, D).astype(jnp.bfloat16)
    b2d = b_fc.reshape(1, 1)
    return _aggregate(node_indexes, A, embedding_states, w1, w2t, b2d)


# speedup vs baseline: 1.1600x; 1.1600x over previous
import jax
import jax.numpy as jnp
from jax import lax
from jax.experimental import pallas as pl
from jax.experimental.pallas import tpu as pltpu

N = 10000
D = 256
B = 1024

_RB = 256
_NBLK = B // _RB


def _body(idx_ref, a_hbm, emb_hbm, emb_ref, w1_ref, w2t_ref, b_ref,
          out_ref, a_buf, cen_buf, t_scr, emb_bf_scr, sems):
    i = pl.program_id(0)
    slot = lax.rem(i, 2)

    def copies(block, slot_):
        descs = []
        for r in range(_RB):
            g = idx_ref[block * _RB + r]
            descs.append(pltpu.make_async_copy(
                a_hbm.at[pl.ds(g, 1)],
                a_buf.at[slot_, pl.ds(r, 1)],
                sems.at[slot_]))
            descs.append(pltpu.make_async_copy(
                emb_hbm.at[pl.ds(g, 1)],
                cen_buf.at[slot_, pl.ds(r, 1)],
                sems.at[slot_]))
        return descs

    def issue(block, slot_):
        for d in copies(block, slot_):
            d.start()

    def drain(block, slot_):
        for d in copies(block, slot_):
            d.wait()

    @pl.when(i == 0)
    def _prologue():
        issue(0, 0)
        emb_bf_scr[...] = emb_ref[...].astype(jnp.bfloat16)
        t_scr[...] = lax.dot_general(
            w2t_ref[...], emb_bf_scr[...], (((1,), (1,)), ((), ())),
            preferred_element_type=jnp.float32) + b_ref[0, 0]

    @pl.when(i + 1 < _NBLK)
    def _prefetch_next():
        issue(i + 1, 1 - slot)

    drain(i, slot)

    cen = cen_buf[slot]
    c_col = jnp.dot(cen, w1_ref[...],
                    preferred_element_type=jnp.float32)
    s = c_col + t_scr[...]
    s = jnp.maximum(s, 0.2 * s)
    w = (a_buf[slot] * s).astype(jnp.bfloat16)
    out_ref[...] = jnp.dot(w, emb_bf_scr[...],
                           preferred_element_type=jnp.float32) + cen


def _aggregate(node_indexes, a, emb, w1, w2t, b2d):
    grid_spec = pltpu.PrefetchScalarGridSpec(
        num_scalar_prefetch=1,
        grid=(_NBLK,),
        in_specs=[
            pl.BlockSpec(memory_space=pl.ANY),
            pl.BlockSpec(memory_space=pl.ANY),
            pl.BlockSpec((N, D), lambda i, idx: (0, 0)),
            pl.BlockSpec((D, 1), lambda i, idx: (0, 0)),
            pl.BlockSpec((1, D), lambda i, idx: (0, 0)),
            pl.BlockSpec((1, 1), lambda i, idx: (0, 0)),
        ],
        out_specs=pl.BlockSpec((_RB, D), lambda i, idx: (i, 0)),
        scratch_shapes=[
            pltpu.VMEM((2, _RB, N), jnp.float32),
            pltpu.VMEM((2, _RB, D), jnp.float32),
            pltpu.VMEM((1, N), jnp.float32),
            pltpu.VMEM((N, D), jnp.bfloat16),
            pltpu.SemaphoreType.DMA((2,)),
        ],
    )
    return pl.pallas_call(
        _body,
        grid_spec=grid_spec,
        out_shape=jax.ShapeDtypeStruct((B, D), jnp.float32),
        compiler_params=pltpu.CompilerParams(
            vmem_limit_bytes=100 * 1024 * 1024),
    )(node_indexes, a, emb, emb, w1, w2t, b2d)


def kernel(node_indexes, A, embedding_states, W_fc, b_fc):
    w1 = W_fc[:D]
    w2t = W_fc[D:].reshape(1, D).astype(jnp.bfloat16)
    b2d = b_fc.reshape(1, 1)
    return _aggregate(node_indexes, A, embedding_states, w1, w2t, b2d)
